# baseline (speedup 1.0000x reference)
import jax
import jax.numpy as jnp
from jax import lax
from jax.experimental import pallas as pl
from jax.experimental.pallas import tpu as pltpu
from jax.experimental.pallas import tpu_sc as plsc

B, N, F = 8, 512, 512
T = 7 * N
NQ = 4
SLOTS = 2
CHUNK = 56
NCHUNK = T // NQ // CHUNK
MAXBLK = 28
VPB = N // 16
VPT = T // 16


def _body(start_hbm, mid_hbm, end_hbm, dur_hbm, out_hbm, mask_hbm,
          dur_v, idx_full, mask_v, rows, zbuf, fixbuf,
          pos_s, row_s, pos_e, row_e, *sems):
    cid = lax.axis_index("c")
    sid = lax.axis_index("s")
    b = cid * 4 + sid // NQ
    q = sid % NQ

    pltpu.sync_copy(dur_hbm.at[b], dur_v)

    lane = lax.iota(jnp.int32, 16)
    row0 = b * N

    def add_fix(pos2d, row2d, cnt, lp, lr, p, nrow, m2):
        own = jnp.logical_and((p // CHUNK) % NQ == q, m2)
        owni = own.astype(jnp.int32)
        pfx = plsc.cumsum(owni) - owni + cnt
        gp = b * T + p
        plsc.store_scatter(pos2d, [pfx // 16, pfx % 16], gp, mask=own)
        plsc.store_scatter(row2d, [pfx // 16, pfx % 16], nrow, mask=own)
        lp = jnp.maximum(lp, jnp.max(jnp.where(own, gp, -1)))
        lr = jnp.maximum(lr, jnp.max(jnp.where(own, nrow, -1)))
        return cnt + jnp.sum(owni), lp, lr

    def seg_body(i, carry):
        cs, lps, lrs, ce, lpe, lre, tot = carry
        d = dur_v[pl.ds(i * 16, 16)]
        o = plsc.cumsum(d) - d + tot
        nrow = row0 + i * 16 + lane
        for j in range(7):
            plsc.store_scatter(idx_full, [o + j], nrow, mask=d > j)
        m2 = d >= 2
        cs, lps, lrs = add_fix(pos_s, row_s, cs, lps, lrs, o, nrow, m2)
        ce, lpe, lre = add_fix(pos_e, row_e, ce, lpe, lre, o + d - 1, nrow, m2)
        return cs, lps, lrs, ce, lpe, lre, tot + jnp.sum(d)

    z = jnp.int32(0)
    cnt_s, lp_s, lr_s, cnt_e, lp_e, lr_e, total = lax.fori_loop(
        0, VPB, seg_body, (z, z, z, z, z, z, z))

    def pad_fix(pos2d, row2d, cnt, lp, lr):
        rem = cnt % 16

        @pl.when(rem != 0)
        def _():
            blk = jnp.full((16,), cnt // 16, jnp.int32)
            m = lane >= rem
            plsc.store_scatter(pos2d, [blk, lane], jnp.full((16,), lp), mask=m)
            plsc.store_scatter(row2d, [blk, lane], jnp.full((16,), lr), mask=m)

    pad_fix(pos_s, row_s, cnt_s, lp_s, lr_s)
    pad_fix(pos_e, row_e, cnt_e, lp_e, lr_e)

    def patch_body(i, _):
        t16 = i * 16 + lane
        v = idx_full[pl.ds(i * 16, 16)]
        idx_full[pl.ds(i * 16, 16)] = jnp.where(t16 >= total, row0, v)
        return _
    patch_hi = jnp.minimum(((total // CHUNK + 1) * CHUNK + 15) // 16, VPT)
    lax.fori_loop(total // 16, patch_hi, patch_body, 0)

    def toff(c):
        return (c * NQ + q) * CHUNK

    def live(c):
        return toff(c) < total

    def gather_start(c, s):
        @pl.when(live(c))
        def _():
            idxs = idx_full.at[pl.ds(toff(c), CHUNK)]
            pltpu.async_copy(mid_hbm.at[idxs], rows.at[s], sems[s])

    def gather_wait(c, s):
        @pl.when(live(c))
        def _():
            idxs = idx_full.at[pl.ds(toff(c), CHUNK)]
            pltpu.make_async_copy(mid_hbm.at[idxs], rows.at[s], sems[s]).wait()

    def store_start(c, s):
        dst = out_hbm.at[pl.ds(b * T + toff(c), CHUNK)]

        @pl.when(live(c))
        def _():
            pltpu.async_copy(rows.at[s], dst, sems[s])

        @pl.when(jnp.logical_not(live(c)))
        def _():
            pltpu.async_copy(zbuf, dst, sems[s])

    def store_wait(c, s):
        dst = out_hbm.at[pl.ds(b * T + toff(c), CHUNK)]
        pltpu.make_async_copy(zbuf, dst, sems[s]).wait()

    def zero_tail(c, s):
        @pl.when(live(c))
        def _():
            lo = jnp.clip(total - toff(c), 0, CHUNK)

            def zrow(r, _):
                for k in range(F // 16):
                    rows[s, r, pl.ds(k * 16, 16)] = jnp.zeros((16,), jnp.float32)
                return _
            lax.fori_loop(lo, CHUNK, zrow, 0)

    for s in range(SLOTS):
        gather_start(s, s)

    def zero_body(r, _):
        for k in range(F // 16):
            zbuf[r, pl.ds(k * 16, 16)] = jnp.zeros((16,), jnp.float32)
        return _
    lax.fori_loop(0, CHUNK, zero_body, 0)

    @pl.when(q == 0)
    def _():
        def mask_body(i, _):
            t16 = i * 16 + lane
            mask_v[pl.ds(i * 16, 16)] = jnp.where(t16 < total, 1, 0)
            return _
        lax.fori_loop(0, VPT, mask_body, 0)
        pltpu.sync_copy(mask_v, mask_hbm.at[b])

    for c in range(NCHUNK):
        s = c % SLOTS
        gather_wait(c, s)
        zero_tail(c, s)
        store_start(c, s)
        cr = c + 1
        if SLOTS <= cr < NCHUNK:
            sr = cr % SLOTS
            store_wait(cr - SLOTS, sr)
            gather_start(cr, sr)
    for c in range(NCHUNK - SLOTS, NCHUNK):
        store_wait(c, c % SLOTS)

    FS = 2

    for src_hbm, pos2d, row2d, cnt in (
            (start_hbm, pos_s, row_s, cnt_s), (end_hbm, pos_e, row_e, cnt_e)):
        def fg(blk, s):
            return pltpu.make_async_copy(
                src_hbm.at[row2d.at[blk]], fixbuf.at[s], sems[s])

        def fs(blk, s):
            return pltpu.make_async_copy(
                fixbuf.at[s], out_hbm.at[pos2d.at[blk]], sems[s])

        for blk in range(MAXBLK):
            s = blk % FS
            lv = blk * 16 < cnt

            @pl.when(lv)
            def _(blk=blk, s=s):
                if blk >= FS:
                    fs(blk - FS, s).wait()
                pltpu.async_copy(
                    src_hbm.at[row2d.at[blk]], fixbuf.at[s], sems[s])
                fg(blk, s).wait()
                pltpu.async_copy(
                    fixbuf.at[s], out_hbm.at[pos2d.at[blk]], sems[s])

        for blk in range(MAXBLK):
            s = blk % FS

            @pl.when(jnp.logical_and(blk * 16 < cnt, (blk + FS) * 16 >= cnt))
            def _(blk=blk, s=s):
                fs(blk, s).wait()


@jax.jit
def _hmm_interp(start2d, mid2d, end2d, durations):
    mesh = plsc.VectorSubcoreMesh(
        core_axis_name="c", subcore_axis_name="s", num_cores=2, num_subcores=16)
    run = pl.kernel(
        _body,
        out_type=(
            jax.ShapeDtypeStruct((B * T, F), jnp.float32),
            jax.ShapeDtypeStruct((B, T), jnp.int32),
        ),
        mesh=mesh,
        scratch_types=[
            pltpu.VMEM((N,), jnp.int32),
            pltpu.VMEM((T,), jnp.int32),
            pltpu.VMEM((T,), jnp.int32),
            pltpu.VMEM((SLOTS, CHUNK, F), jnp.float32),
            pltpu.VMEM((CHUNK, F), jnp.float32),
            pltpu.VMEM((2, 16, F), jnp.float32),
            pltpu.VMEM((MAXBLK, 16), jnp.int32),
            pltpu.VMEM((MAXBLK, 16), jnp.int32),
            pltpu.VMEM((MAXBLK, 16), jnp.int32),
            pltpu.VMEM((MAXBLK, 16), jnp.int32),
        ] + [pltpu.SemaphoreType.DMA] * SLOTS,
        compiler_params=pltpu.CompilerParams(needs_layout_passes=False),
    )
    return run(start2d, mid2d, end2d, durations)


def kernel(start, mid, end, durations, max_frames):
    out_flat, mask_i32 = _hmm_interp(
        start.reshape(B * N, F), mid.reshape(B * N, F),
        end.reshape(B * N, F), durations)
    return out_flat.reshape(B, T, F), mask_i32.astype(jnp.bool_)


# speedup vs baseline: 1.0145x; 1.0145x over previous
import jax
import jax.numpy as jnp
from jax import lax
from jax.experimental import pallas as pl
from jax.experimental.pallas import tpu as pltpu
from jax.experimental.pallas import tpu_sc as plsc

B, N, F = 8, 512, 512
T = 7 * N
NQ = 4
SLOTS = 2
CHUNK = 56
NCHUNK = T // NQ // CHUNK
MAXBLK = 28
VPB = N // 16
VPT = T // 16


def _body(start_hbm, mid_hbm, end_hbm, dur_hbm, out_hbm, mask_hbm,
          dur_v, idx_full, mask_v, rows, zbuf, fixbuf,
          pos_s, row_s, pos_e, row_e, *sems):
    cid = lax.axis_index("c")
    sid = lax.axis_index("s")
    b = cid * 4 + sid // NQ
    q = sid % NQ

    pltpu.sync_copy(dur_hbm.at[b], dur_v)

    lane = lax.iota(jnp.int32, 16)
    row0 = b * N

    def add_fix(pos2d, row2d, cnt, lp, lr, p, nrow, m2):
        own = jnp.logical_and((p // CHUNK) % NQ == q, m2)
        owni = own.astype(jnp.int32)
        pfx = plsc.cumsum(owni) - owni + cnt
        gp = b * T + p
        plsc.store_scatter(pos2d, [pfx // 16, pfx % 16], gp, mask=own)
        plsc.store_scatter(row2d, [pfx // 16, pfx % 16], nrow, mask=own)
        lp = jnp.maximum(lp, jnp.max(jnp.where(own, gp, -1)))
        lr = jnp.maximum(lr, jnp.max(jnp.where(own, nrow, -1)))
        return cnt + jnp.sum(owni), lp, lr

    def seg_body(i, carry):
        cs, lps, lrs, ce, lpe, lre, tot = carry
        d = dur_v[pl.ds(i * 16, 16)]
        o = plsc.cumsum(d) - d + tot
        nrow = row0 + i * 16 + lane
        for j in range(7):
            plsc.store_scatter(idx_full, [o + j], nrow, mask=d > j)
        m2 = d >= 2
        cs, lps, lrs = add_fix(pos_s, row_s, cs, lps, lrs, o, nrow, m2)
        ce, lpe, lre = add_fix(pos_e, row_e, ce, lpe, lre, o + d - 1, nrow, m2)
        return cs, lps, lrs, ce, lpe, lre, tot + jnp.sum(d)

    z = jnp.int32(0)
    cnt_s, lp_s, lr_s, cnt_e, lp_e, lr_e, total = lax.fori_loop(
        0, VPB, seg_body, (z, z, z, z, z, z, z))

    def pad_fix(pos2d, row2d, cnt, lp, lr):
        rem = cnt % 16

        @pl.when(rem != 0)
        def _():
            blk = jnp.full((16,), cnt // 16, jnp.int32)
            m = lane >= rem
            plsc.store_scatter(pos2d, [blk, lane], jnp.full((16,), lp), mask=m)
            plsc.store_scatter(row2d, [blk, lane], jnp.full((16,), lr), mask=m)

    pad_fix(pos_s, row_s, cnt_s, lp_s, lr_s)
    pad_fix(pos_e, row_e, cnt_e, lp_e, lr_e)

    def patch_body(i, _):
        t16 = i * 16 + lane
        v = idx_full[pl.ds(i * 16, 16)]
        idx_full[pl.ds(i * 16, 16)] = jnp.where(t16 >= total, row0, v)
        return _
    patch_hi = jnp.minimum(((total // CHUNK + 1) * CHUNK + 15) // 16, VPT)
    lax.fori_loop(total // 16, patch_hi, patch_body, 0)

    def toff(c):
        return (c * NQ + q) * CHUNK

    def live(c):
        return toff(c) < total

    def gather_start(c, s):
        @pl.when(live(c))
        def _():
            idxs = idx_full.at[pl.ds(toff(c), CHUNK)]
            pltpu.async_copy(mid_hbm.at[idxs], rows.at[s], sems[s])

    def gather_wait(c, s):
        @pl.when(live(c))
        def _():
            idxs = idx_full.at[pl.ds(toff(c), CHUNK)]
            pltpu.make_async_copy(mid_hbm.at[idxs], rows.at[s], sems[s]).wait()

    def store_start(c, s):
        dst = out_hbm.at[pl.ds(b * T + toff(c), CHUNK)]

        @pl.when(live(c))
        def _():
            pltpu.async_copy(rows.at[s], dst, sems[s])

        @pl.when(jnp.logical_not(live(c)))
        def _():
            pltpu.async_copy(zbuf, dst, sems[s])

    def store_wait(c, s):
        dst = out_hbm.at[pl.ds(b * T + toff(c), CHUNK)]
        pltpu.make_async_copy(zbuf, dst, sems[s]).wait()

    def zero_tail(c, s):
        @pl.when(live(c))
        def _():
            lo = jnp.clip(total - toff(c), 0, CHUNK)

            def zrow(r, _):
                for k in range(F // 16):
                    rows[s, r, pl.ds(k * 16, 16)] = jnp.zeros((16,), jnp.float32)
                return _
            lax.fori_loop(lo, CHUNK, zrow, 0)

    for s in range(SLOTS):
        gather_start(s, s)

    def zero_body(r, _):
        for k in range(F // 16):
            zbuf[r, pl.ds(k * 16, 16)] = jnp.zeros((16,), jnp.float32)
        return _
    lax.fori_loop(0, CHUNK, zero_body, 0)

    @pl.when(q == 0)
    def _():
        def mask_body(i, _):
            t16 = i * 16 + lane
            mask_v[pl.ds(i * 16, 16)] = jnp.where(t16 < total, 1, 0)
            return _
        lax.fori_loop(0, VPT, mask_body, 0)
        pltpu.sync_copy(mask_v, mask_hbm.at[b])

    for c in range(NCHUNK):
        s = c % SLOTS
        gather_wait(c, s)
        zero_tail(c, s)
        store_start(c, s)
        cr = c + 1
        if SLOTS <= cr < NCHUNK:
            sr = cr % SLOTS
            store_wait(cr - SLOTS, sr)
            gather_start(cr, sr)
    for c in range(NCHUNK - SLOTS, NCHUNK):
        store_wait(c, c % SLOTS)

    FS = 2
    blocks = [(start_hbm, pos_s, row_s, 0, blk) for blk in range(MAXBLK)] + \
             [(end_hbm, pos_e, row_e, 1, blk) for blk in range(MAXBLK)]
    NB = len(blocks)

    def flive(k):
        _, _, _, li, blk = blocks[k]
        return blk * 16 < (cnt_s if li == 0 else cnt_e)

    def fgather_start(k, s):
        src_hbm, _, row2d, _, blk = blocks[k]

        @pl.when(flive(k))
        def _():
            pltpu.async_copy(src_hbm.at[row2d.at[blk]], fixbuf.at[s], sems[s])

    def fgather_wait(k, s):
        src_hbm, _, row2d, _, blk = blocks[k]

        @pl.when(flive(k))
        def _():
            pltpu.make_async_copy(
                src_hbm.at[row2d.at[blk]], fixbuf.at[s], sems[s]).wait()

    def fscatter_start(k, s):
        _, pos2d, _, _, blk = blocks[k]

        @pl.when(flive(k))
        def _():
            pltpu.async_copy(fixbuf.at[s], out_hbm.at[pos2d.at[blk]], sems[s])

    def fscatter_wait(k, s):
        _, pos2d, _, _, blk = blocks[k]

        @pl.when(flive(k))
        def _():
            pltpu.make_async_copy(
                fixbuf.at[s], out_hbm.at[pos2d.at[blk]], sems[s]).wait()

    for s in range(FS):
        fgather_start(s, s)
    for k in range(NB):
        s = k % FS
        fgather_wait(k, s)
        fscatter_start(k, s)
        kr = k + 1
        if FS <= kr < NB:
            sr = kr % FS
            fscatter_wait(kr - FS, sr)
            fgather_start(kr, sr)
    for k in range(NB - FS, NB):
        fscatter_wait(k, k % FS)


@jax.jit
def _hmm_interp(start2d, mid2d, end2d, durations):
    mesh = plsc.VectorSubcoreMesh(
        core_axis_name="c", subcore_axis_name="s", num_cores=2, num_subcores=16)
    run = pl.kernel(
        _body,
        out_type=(
            jax.ShapeDtypeStruct((B * T, F), jnp.float32),
            jax.ShapeDtypeStruct((B, T), jnp.int32),
        ),
        mesh=mesh,
        scratch_types=[
            pltpu.VMEM((N,), jnp.int32),
            pltpu.VMEM((T,), jnp.int32),
            pltpu.VMEM((T,), jnp.int32),
            pltpu.VMEM((SLOTS, CHUNK, F), jnp.float32),
            pltpu.VMEM((CHUNK, F), jnp.float32),
            pltpu.VMEM((2, 16, F), jnp.float32),
            pltpu.VMEM((MAXBLK, 16), jnp.int32),
            pltpu.VMEM((MAXBLK, 16), jnp.int32),
            pltpu.VMEM((MAXBLK, 16), jnp.int32),
            pltpu.VMEM((MAXBLK, 16), jnp.int32),
        ] + [pltpu.SemaphoreType.DMA] * SLOTS,
        compiler_params=pltpu.CompilerParams(needs_layout_passes=False),
    )
    return run(start2d, mid2d, end2d, durations)


def kernel(start, mid, end, durations, max_frames):
    out_flat, mask_i32 = _hmm_interp(
        start.reshape(B * N, F), mid.reshape(B * N, F),
        end.reshape(B * N, F), durations)
    return out_flat.reshape(B, T, F), mask_i32.astype(jnp.bool_)


# speedup vs baseline: 1.2151x; 1.1978x over previous
import functools

import jax
import jax.numpy as jnp
from jax import lax
from jax.experimental import pallas as pl
from jax.experimental.pallas import tpu as pltpu
from jax.experimental.pallas import tpu_sc as plsc

B, N, F = 8, 512, 512
T = 7 * N
TBL_ROWS = B * 3 * N
NQ = 4
TQ = T // NQ
SLOTS = 3
CHUNK = 56
NCHUNK = TQ // CHUNK
VPB = N // 16
VPT = T // 16


def _body(tbl_hbm, dur_hbm, out_hbm, mask_hbm, dur_v, idx_full, mask_v, rows, zbuf, *sems):
    cid = lax.axis_index("c")
    sid = lax.axis_index("s")
    b = cid * 4 + sid // NQ
    q = sid % NQ

    pltpu.sync_copy(dur_hbm.at[b], dur_v)

    base_b = b * (3 * N)
    lane = lax.iota(jnp.int32, 16)

    def seg_body(i, carry):
        d = dur_v[pl.ds(i * 16, 16)]
        o = plsc.cumsum(d) - d + carry
        n = base_b + i * 16 + lane
        vmid = n + N
        for j in range(7):
            if j == 0:
                val = jnp.where(d >= 2, n, vmid)
            else:
                val = jnp.where(d == j + 1, n + 2 * N, vmid)
            plsc.store_scatter(idx_full, [o + j], val, mask=d > j)
        return carry + jnp.sum(d)
    total = lax.fori_loop(0, VPB, seg_body, jnp.int32(0))

    def patch_body(i, _):
        t16 = i * 16 + lane
        v = idx_full[pl.ds(i * 16, 16)]
        idx_full[pl.ds(i * 16, 16)] = jnp.where(t16 >= total, base_b, v)
        return _
    patch_hi = jnp.minimum(((total // CHUNK + 1) * CHUNK + 15) // 16, VPT)
    lax.fori_loop(total // 16, patch_hi, patch_body, 0)

    def toff(c):
        return (c * NQ + q) * CHUNK

    def live(c):
        return toff(c) < total

    def gather_start(c, s):
        @pl.when(live(c))
        def _():
            idxs = idx_full.at[pl.ds(toff(c), CHUNK)]
            pltpu.async_copy(tbl_hbm.at[idxs], rows.at[s], sems[s])

    def gather_wait(c, s):
        @pl.when(live(c))
        def _():
            idxs = idx_full.at[pl.ds(toff(c), CHUNK)]
            pltpu.make_async_copy(tbl_hbm.at[idxs], rows.at[s], sems[s]).wait()

    def store_start(c, s):
        dst = out_hbm.at[pl.ds(b * T + toff(c), CHUNK)]

        @pl.when(live(c))
        def _():
            pltpu.async_copy(rows.at[s], dst, sems[s])

        @pl.when(jnp.logical_not(live(c)))
        def _():
            pltpu.async_copy(zbuf, dst, sems[s])

    def store_wait(c, s):
        dst = out_hbm.at[pl.ds(b * T + toff(c), CHUNK)]
        pltpu.make_async_copy(zbuf, dst, sems[s]).wait()

    for s in range(SLOTS):
        gather_start(s, s)

    def zero_body(r, _):
        for k in range(F // 16):
            zbuf[r, pl.ds(k * 16, 16)] = jnp.zeros((16,), jnp.float32)
        return _
    lax.fori_loop(0, CHUNK, zero_body, 0)

    @pl.when(q == 0)
    def _():
        def mask_body(i, _):
            t16 = i * 16 + lane
            mask_v[pl.ds(i * 16, 16)] = jnp.where(t16 < total, 1, 0)
            return _
        lax.fori_loop(0, VPT, mask_body, 0)
        pltpu.sync_copy(mask_v, mask_hbm.at[b])

    def zero_tail(c, s):
        @pl.when(live(c))
        def _():
            lo = jnp.clip(total - toff(c), 0, CHUNK)

            def zrow(r, _):
                for k in range(F // 16):
                    rows[s, r, pl.ds(k * 16, 16)] = jnp.zeros((16,), jnp.float32)
                return _
            lax.fori_loop(lo, CHUNK, zrow, 0)

    for c in range(NCHUNK):
        s = c % SLOTS
        gather_wait(c, s)
        zero_tail(c, s)
        store_start(c, s)
        cr = c + 1
        if SLOTS <= cr < NCHUNK:
            sr = cr % SLOTS
            store_wait(cr - SLOTS, sr)
            gather_start(cr, sr)
    for c in range(NCHUNK - SLOTS, NCHUNK):
        store_wait(c, c % SLOTS)


@jax.jit
def _hmm_interp(table, durations):
    mesh = plsc.VectorSubcoreMesh(
        core_axis_name="c", subcore_axis_name="s", num_cores=2, num_subcores=16)
    run = pl.kernel(
        _body,
        out_type=(
            jax.ShapeDtypeStruct((B * T, F), jnp.float32),
            jax.ShapeDtypeStruct((B, T), jnp.int32),
        ),
        mesh=mesh,
        scratch_types=[
            pltpu.VMEM((N,), jnp.int32),
            pltpu.VMEM((T,), jnp.int32),
            pltpu.VMEM((T,), jnp.int32),
            pltpu.VMEM((SLOTS, CHUNK, F), jnp.float32),
            pltpu.VMEM((CHUNK, F), jnp.float32),
        ] + [pltpu.SemaphoreType.DMA] * SLOTS,
        compiler_params=pltpu.CompilerParams(needs_layout_passes=False),
    )
    return run(table, durations)


def kernel(start, mid, end, durations, max_frames):
    table = jnp.concatenate([start, mid, end], axis=1).reshape(B * 3 * N, F)
    out_flat, mask_i32 = _hmm_interp(table, durations)
    return out_flat.reshape(B, T, F), mask_i32.astype(jnp.bool_)


# speedup vs baseline: 1.2396x; 1.0201x over previous
import functools

import jax
import jax.numpy as jnp
from jax import lax
from jax.experimental import pallas as pl
from jax.experimental.pallas import tpu as pltpu
from jax.experimental.pallas import tpu_sc as plsc

B, N, F = 8, 512, 512
T = 7 * N
TBL_ROWS = B * 3 * N
NQ = 4
TQ = T // NQ
SLOTS = 6
CHUNK = 32
NCHUNK = TQ // CHUNK
VPB = N // 16
VPT = T // 16


def _body(tbl_hbm, dur_hbm, out_hbm, mask_hbm, dur_v, idx_full, mask_v, rows, zbuf, *sems):
    cid = lax.axis_index("c")
    sid = lax.axis_index("s")
    b = cid * 4 + sid // NQ
    q = sid % NQ

    pltpu.sync_copy(dur_hbm.at[b], dur_v)

    base_b = b * (3 * N)
    lane = lax.iota(jnp.int32, 16)

    def seg_body(i, carry):
        d = dur_v[pl.ds(i * 16, 16)]
        o = plsc.cumsum(d) - d + carry
        n = base_b + i * 16 + lane
        vmid = n + N
        for j in range(7):
            if j == 0:
                val = jnp.where(d >= 2, n, vmid)
            else:
                val = jnp.where(d == j + 1, n + 2 * N, vmid)
            plsc.store_scatter(idx_full, [o + j], val, mask=d > j)
        return carry + jnp.sum(d)
    total = lax.fori_loop(0, VPB, seg_body, jnp.int32(0))

    def patch_body(i, _):
        t16 = i * 16 + lane
        v = idx_full[pl.ds(i * 16, 16)]
        idx_full[pl.ds(i * 16, 16)] = jnp.where(t16 >= total, base_b, v)
        return _
    patch_hi = jnp.minimum(((total // CHUNK + 1) * CHUNK + 15) // 16, VPT)
    lax.fori_loop(total // 16, patch_hi, patch_body, 0)

    def toff(c):
        return (c * NQ + q) * CHUNK

    def live(c):
        return toff(c) < total

    def gather_start(c, s):
        @pl.when(live(c))
        def _():
            idxs = idx_full.at[pl.ds(toff(c), CHUNK)]
            pltpu.async_copy(tbl_hbm.at[idxs], rows.at[s], sems[s])

    def gather_wait(c, s):
        @pl.when(live(c))
        def _():
            idxs = idx_full.at[pl.ds(toff(c), CHUNK)]
            pltpu.make_async_copy(tbl_hbm.at[idxs], rows.at[s], sems[s]).wait()

    def store_start(c, s):
        dst = out_hbm.at[pl.ds(b * T + toff(c), CHUNK)]

        @pl.when(live(c))
        def _():
            pltpu.async_copy(rows.at[s], dst, sems[s])

        @pl.when(jnp.logical_not(live(c)))
        def _():
            pltpu.async_copy(zbuf, dst, sems[s])

    def store_wait(c, s):
        dst = out_hbm.at[pl.ds(b * T + toff(c), CHUNK)]
        pltpu.make_async_copy(zbuf, dst, sems[s]).wait()

    for s in range(SLOTS):
        gather_start(s, s)

    def zero_body(r, _):
        for k in range(F // 16):
            zbuf[r, pl.ds(k * 16, 16)] = jnp.zeros((16,), jnp.float32)
        return _
    lax.fori_loop(0, CHUNK, zero_body, 0)

    @pl.when(q == 0)
    def _():
        def mask_body(i, _):
            t16 = i * 16 + lane
            mask_v[pl.ds(i * 16, 16)] = jnp.where(t16 < total, 1, 0)
            return _
        lax.fori_loop(0, VPT, mask_body, 0)
        pltpu.sync_copy(mask_v, mask_hbm.at[b])

    def zero_tail(c, s):
        @pl.when(live(c))
        def _():
            lo = jnp.clip(total - toff(c), 0, CHUNK)

            def zrow(r, _):
                for k in range(F // 16):
                    rows[s, r, pl.ds(k * 16, 16)] = jnp.zeros((16,), jnp.float32)
                return _
            lax.fori_loop(lo, CHUNK, zrow, 0)

    for c in range(NCHUNK):
        s = c % SLOTS
        gather_wait(c, s)
        zero_tail(c, s)
        store_start(c, s)
        cr = c + 1
        if SLOTS <= cr < NCHUNK:
            sr = cr % SLOTS
            store_wait(cr - SLOTS, sr)
            gather_start(cr, sr)
    for c in range(NCHUNK - SLOTS, NCHUNK):
        store_wait(c, c % SLOTS)


@jax.jit
def _hmm_interp(table, durations):
    mesh = plsc.VectorSubcoreMesh(
        core_axis_name="c", subcore_axis_name="s", num_cores=2, num_subcores=16)
    run = pl.kernel(
        _body,
        out_type=(
            jax.ShapeDtypeStruct((B * T, F), jnp.float32),
            jax.ShapeDtypeStruct((B, T), jnp.int32),
        ),
        mesh=mesh,
        scratch_types=[
            pltpu.VMEM((N,), jnp.int32),
            pltpu.VMEM((T,), jnp.int32),
            pltpu.VMEM((T,), jnp.int32),
            pltpu.VMEM((SLOTS, CHUNK, F), jnp.float32),
            pltpu.VMEM((CHUNK, F), jnp.float32),
        ] + [pltpu.SemaphoreType.DMA] * SLOTS,
        compiler_params=pltpu.CompilerParams(needs_layout_passes=False),
    )
    return run(table, durations)


def kernel(start, mid, end, durations, max_frames):
    table = jnp.concatenate([start, mid, end], axis=1).reshape(B * 3 * N, F)
    out_flat, mask_i32 = _hmm_interp(table, durations)
    return out_flat.reshape(B, T, F), mask_i32.astype(jnp.bool_)
